# baseline (speedup 1.0000x reference)
import jax
import jax.numpy as jnp
from jax import lax
from jax.experimental import pallas as pl
from jax.experimental.pallas import tpu as pltpu
from jax.experimental.pallas import tpu_sc as plsc

_B = 64
_V = 1_000_000
_LANES = 16
_CW = 128
_NCHUNK = _B * _V // _CW
_VB = 8192
_NBLK = (_V + _VB - 1) // _VB


_TAIL = (_V // _CW) * _CW
_TAILW = _V - _TAIL


def _sc_gather_body(x_hbm, cls_hbm, out_hbm, tail_hbm, cls_v, rows_v, tail_v,
                    sem):
    wid = lax.axis_index("s") * 2 + lax.axis_index("c")

    @pl.when(wid < _B // _LANES)
    def _():
        base = wid * _LANES
        pltpu.sync_copy(cls_hbm.at[pl.ds(base, _LANES)], cls_v)
        starts = jnp.minimum((cls_v[...] >> 7) << 7, _TAIL - _CW)
        descs = []
        for i in range(_LANES):
            rstart = pl.multiple_of(base + (i // 8) * 8, 8)
            cstart = pl.multiple_of(starts[i], _CW)
            descs.append(pltpu.async_copy(
                x_hbm.at[pl.ds(rstart, 8), pl.ds(cstart, _CW)],
                rows_v.at[i], sem))
        for d in descs:
            d.wait()
        pltpu.sync_copy(rows_v, out_hbm.at[pl.ds(base, _LANES)])
        pltpu.sync_copy(x_hbm.at[pl.ds(base, _LANES), pl.ds(_TAIL, _TAILW)],
                        tail_v)
        pltpu.sync_copy(tail_v, tail_hbm.at[pl.ds(base, _LANES)])


def _sc_gather(x, cls):
    mesh = plsc.VectorSubcoreMesh(core_axis_name="c", subcore_axis_name="s")
    return pl.kernel(
        _sc_gather_body,
        mesh=mesh,
        out_type=(jax.ShapeDtypeStruct((_B, 8, _CW), jnp.float32),
                  jax.ShapeDtypeStruct((_B, _TAILW), jnp.float32)),
        scratch_types=[
            pltpu.VMEM((_LANES,), jnp.int32),
            pltpu.VMEM((_LANES, 8, _CW), jnp.float32),
            pltpu.VMEM((_LANES, _TAILW), jnp.float32),
            pltpu.SemaphoreType.DMA,
        ],
    )(x, cls)


def _count_body(rows_ref, tail_ref, cls_ref, x_ref, top1_ref, top5_ref,
                acc_ref):
    i = pl.program_id(0)

    @pl.when(i == 0)
    def _():
        acc_ref[...] = jnp.zeros_like(acc_ref)

    vals = x_ref[...]
    c = cls_ref[...]
    c3 = c.reshape(_B, 1, 1)
    brow3 = lax.broadcasted_iota(jnp.int32, (_B, 8, _CW), 0)
    sub3 = lax.broadcasted_iota(jnp.int32, (_B, 8, _CW), 1)
    lane3 = lax.broadcasted_iota(jnp.int32, (_B, 8, _CW), 2)
    hot = (sub3 == (brow3 & 7)) & (lane3 == (c3 & (_CW - 1)))
    xc_slab = jnp.sum(jnp.where(hot, rows_ref[...], 0.0), axis=(1, 2))
    hot_t = lax.broadcasted_iota(jnp.int32, (_B, _TAILW), 1) == (c - _TAIL)
    xc_tail = jnp.sum(jnp.where(hot_t, tail_ref[...], 0.0), axis=1)
    xc = jnp.where(c[:, 0] >= _TAIL, xc_tail, xc_slab).reshape(_B, 1)
    col = lax.broadcasted_iota(jnp.int32, (_B, _VB), 1) + i * _VB
    gt = (vals > xc) & (col < _V)
    eqb = (vals == xc) & (col < c)
    acc_ref[...] = acc_ref[...] + (gt | eqb).astype(jnp.int32)

    @pl.when(i == _NBLK - 1)
    def _():
        rank = jnp.sum(acc_ref[...], axis=1, keepdims=True)
        top1_ref[...] = jnp.sum((rank == 0).astype(jnp.int32), keepdims=True)
        top5_ref[...] = jnp.sum((rank < 5).astype(jnp.int32), keepdims=True)


def _tc_count(x, rows, tail, cls):
    return pl.pallas_call(
        _count_body,
        grid=(_NBLK,),
        in_specs=[
            pl.BlockSpec((_B, 8, _CW), lambda i: (0, 0, 0)),
            pl.BlockSpec((_B, _TAILW), lambda i: (0, 0)),
            pl.BlockSpec((_B, 1), lambda i: (0, 0)),
            pl.BlockSpec((_B, _VB), lambda i: (0, i)),
        ],
        out_specs=[
            pl.BlockSpec((1, 1), lambda i: (0, 0)),
            pl.BlockSpec((1, 1), lambda i: (0, 0)),
        ],
        out_shape=[
            jax.ShapeDtypeStruct((1, 1), jnp.int32),
            jax.ShapeDtypeStruct((1, 1), jnp.int32),
        ],
        scratch_shapes=[pltpu.VMEM((_B, _VB), jnp.int32)],
        compiler_params=pltpu.CompilerParams(
            dimension_semantics=("arbitrary",)),
    )(rows, tail, cls, x)


def kernel(x, classes):
    cls = classes.astype(jnp.int32).reshape(_B)
    rows, tail = _sc_gather(x, cls)
    top1, top5 = _tc_count(x, rows, tail, cls.reshape(_B, 1))
    return top1[0, 0], top5[0, 0]


# speedup vs baseline: 1.3708x; 1.3708x over previous
import jax
import jax.numpy as jnp
from jax import lax
from jax.experimental import pallas as pl
from jax.experimental.pallas import tpu as pltpu
from jax.experimental.pallas import tpu_sc as plsc

_B = 64
_V = 1_000_000
_LANES = 16
_CW = 128
_NCHUNK = _B * _V // _CW
_VB = 16384
_NBLK = (_V + _VB - 1) // _VB
_LASTW = _V - (_NBLK - 1) * _VB


_TAIL = (_V // _CW) * _CW
_TAILW = _V - _TAIL


def _sc_gather_body(x_hbm, cls_hbm, out_hbm, tail_hbm, cls_v, rows_v, tail_v,
                    sem):
    wid = lax.axis_index("s") * 2 + lax.axis_index("c")

    @pl.when(wid < _B // _LANES)
    def _():
        base = wid * _LANES
        pltpu.sync_copy(cls_hbm.at[pl.ds(base, _LANES)], cls_v)
        starts = jnp.minimum((cls_v[...] >> 7) << 7, _TAIL - _CW)
        descs = []
        for i in range(_LANES):
            rstart = pl.multiple_of(base + (i // 8) * 8, 8)
            cstart = pl.multiple_of(starts[i], _CW)
            descs.append(pltpu.async_copy(
                x_hbm.at[pl.ds(rstart, 8), pl.ds(cstart, _CW)],
                rows_v.at[i], sem))
        for d in descs:
            d.wait()
        pltpu.sync_copy(rows_v, out_hbm.at[pl.ds(base, _LANES)])
        pltpu.sync_copy(x_hbm.at[pl.ds(base, _LANES), pl.ds(_TAIL, _TAILW)],
                        tail_v)
        pltpu.sync_copy(tail_v, tail_hbm.at[pl.ds(base, _LANES)])


def _sc_gather(x, cls):
    mesh = plsc.VectorSubcoreMesh(core_axis_name="c", subcore_axis_name="s")
    return pl.kernel(
        _sc_gather_body,
        mesh=mesh,
        out_type=(jax.ShapeDtypeStruct((_B, 8, _CW), jnp.float32),
                  jax.ShapeDtypeStruct((_B, _TAILW), jnp.float32)),
        scratch_types=[
            pltpu.VMEM((_LANES,), jnp.int32),
            pltpu.VMEM((_LANES, 8, _CW), jnp.float32),
            pltpu.VMEM((_LANES, _TAILW), jnp.float32),
            pltpu.SemaphoreType.DMA,
        ],
    )(x, cls)


def _count_body(rows_ref, tail_ref, cls_ref, x_ref, top1_ref, top5_ref,
                acc_ref, xc_ref, lane_ref):
    i = pl.program_id(0)
    c = cls_ref[...]

    @pl.when(i == 0)
    def _():
        acc_ref[...] = jnp.zeros_like(acc_ref)
        lane_ref[...] = lax.broadcasted_iota(jnp.int32, (_B, _VB), 1)
        c3 = c.reshape(_B, 1, 1)
        brow3 = lax.broadcasted_iota(jnp.int32, (_B, 8, _CW), 0)
        sub3 = lax.broadcasted_iota(jnp.int32, (_B, 8, _CW), 1)
        lane3 = lax.broadcasted_iota(jnp.int32, (_B, 8, _CW), 2)
        hot = (sub3 == (brow3 & 7)) & (lane3 == (c3 & (_CW - 1)))
        xc_slab = jnp.sum(jnp.where(hot, rows_ref[...], 0.0), axis=(1, 2))
        hot_t = lax.broadcasted_iota(jnp.int32, (_B, _TAILW), 1) == (c - _TAIL)
        xc_tail = jnp.sum(jnp.where(hot_t, tail_ref[...], 0.0), axis=1)
        xc_ref[...] = jnp.where(c[:, 0] >= _TAIL, xc_tail,
                                xc_slab).reshape(_B, 1)

    vals = x_ref[...]
    xc = xc_ref[...]
    lanes = lane_ref[...]
    eqb = (vals == xc) & (lanes < (c - i * _VB))

    @pl.when(i < _NBLK - 1)
    def _():
        acc_ref[...] = acc_ref[...] + ((vals > xc) | eqb).astype(jnp.int32)

    @pl.when(i == _NBLK - 1)
    def _():
        gt = (vals > xc) & (lanes < _LASTW)
        acc_ref[...] = acc_ref[...] + (gt | eqb).astype(jnp.int32)
        rank = jnp.sum(acc_ref[...], axis=1, keepdims=True)
        top1_ref[...] = jnp.sum((rank == 0).astype(jnp.int32), keepdims=True)
        top5_ref[...] = jnp.sum((rank < 5).astype(jnp.int32), keepdims=True)


def _tc_count(x, rows, tail, cls):
    return pl.pallas_call(
        _count_body,
        grid=(_NBLK,),
        in_specs=[
            pl.BlockSpec((_B, 8, _CW), lambda i: (0, 0, 0)),
            pl.BlockSpec((_B, _TAILW), lambda i: (0, 0)),
            pl.BlockSpec((_B, 1), lambda i: (0, 0)),
            pl.BlockSpec((_B, _VB), lambda i: (0, i)),
        ],
        out_specs=[
            pl.BlockSpec((1, 1), lambda i: (0, 0)),
            pl.BlockSpec((1, 1), lambda i: (0, 0)),
        ],
        out_shape=[
            jax.ShapeDtypeStruct((1, 1), jnp.int32),
            jax.ShapeDtypeStruct((1, 1), jnp.int32),
        ],
        scratch_shapes=[
            pltpu.VMEM((_B, _VB), jnp.int32),
            pltpu.VMEM((_B, 1), jnp.float32),
            pltpu.VMEM((_B, _VB), jnp.int32),
        ],
        compiler_params=pltpu.CompilerParams(
            dimension_semantics=("arbitrary",)),
    )(rows, tail, cls, x)


def kernel(x, classes):
    cls = classes.astype(jnp.int32).reshape(_B)
    rows, tail = _sc_gather(x, cls)
    top1, top5 = _tc_count(x, rows, tail, cls.reshape(_B, 1))
    return top1[0, 0], top5[0, 0]
